# baseline (speedup 1.0000x reference)
import functools

import jax
import jax.numpy as jnp
from jax import lax
from jax.experimental import pallas as pl
from jax.experimental.pallas import tpu as pltpu
from jax.experimental.pallas import tpu_sc as plsc

VOCAB = 100000
EMB_DIM = 64
BATCH = 1024
CTX = 10

NC, NS = 2, 16
NW = NC * NS
BPW = BATCH // NW
IPW = BPW * CTX
IPW_PAD = 384
NCHUNK = IPW_PAD // 128

VT = 2048
NV = (VOCAB + VT - 1) // VT
NEG = -1e30


def _sc_gather_sum(xp, emb):
    mesh = plsc.VectorSubcoreMesh(core_axis_name="c", subcore_axis_name="s")

    @functools.partial(
        pl.kernel,
        mesh=mesh,
        compiler_params=pltpu.CompilerParams(use_tc_tiling_on_sc=False),
        out_type=jax.ShapeDtypeStruct((BATCH, EMB_DIM), jnp.float32),
        scratch_types=[
            pltpu.VMEM((NCHUNK, 128), jnp.int32),
            pltpu.VMEM((IPW_PAD, EMB_DIM), jnp.float32),
            pltpu.VMEM((BPW, EMB_DIM), jnp.float32),
            pltpu.SemaphoreType.DMA,
        ],
    )
    def k(xp_hbm, emb_hbm, out_hbm, idx_v, rows_v, acc_v, sem):
        wid = lax.axis_index("s") * NC + lax.axis_index("c")
        pltpu.sync_copy(xp_hbm.at[wid], idx_v)
        copies = [
            pltpu.async_copy(
                emb_hbm.at[idx_v.at[c]],
                rows_v.at[pl.ds(c * 128, 128)],
                sem,
            )
            for c in range(NCHUNK)
        ]
        for cp in copies:
            cp.wait()
        for bi in range(BPW):
            for c4 in range(EMB_DIM // 16):
                sl = pl.ds(c4 * 16, 16)
                acc = rows_v[bi * CTX, sl]
                for j in range(1, CTX):
                    acc = acc + rows_v[bi * CTX + j, sl]
                acc_v[bi, sl] = acc
        pltpu.sync_copy(acc_v, out_hbm.at[pl.ds(wid * BPW, BPW)])

    return k(xp, emb)


def _tc_body(s_ref, w_ref, b_ref, out_ref, m_ref, l_ref, lse_ref):
    p = pl.program_id(0)
    v = pl.program_id(1)
    logits = lax.dot_general(
        s_ref[...], w_ref[...],
        (((1,), (1,)), ((), ())),
        preferred_element_type=jnp.float32,
    ) + b_ref[...]

    @pl.when(p == 0)
    def _phase0():
        col = lax.broadcasted_iota(jnp.int32, (1, VT), 1)
        lg = jnp.where(col < (VOCAB - v * VT), logits, NEG)

        @pl.when(v == 0)
        def _init():
            m_ref[...] = jnp.full((BATCH, 1), NEG, jnp.float32)
            l_ref[...] = jnp.zeros((BATCH, 1), jnp.float32)

        tmax = jnp.max(lg, axis=1, keepdims=True)
        m_new = jnp.maximum(m_ref[...], tmax)
        l_ref[...] = (l_ref[...] * jnp.exp(m_ref[...] - m_new)
                      + jnp.sum(jnp.exp(lg - m_new), axis=1, keepdims=True))
        m_ref[...] = m_new

        @pl.when(v == NV - 1)
        def _fin():
            lse_ref[...] = m_ref[...] + jnp.log(l_ref[...])

    @pl.when(p == 1)
    def _phase1():
        out_ref[...] = logits - lse_ref[...]


def _tc_logsoftmax(s, W, b2, interpret=False):
    return pl.pallas_call(
        _tc_body,
        grid=(2, NV),
        in_specs=[
            pl.BlockSpec((BATCH, EMB_DIM), lambda p, v: (0, 0)),
            pl.BlockSpec((VT, EMB_DIM), lambda p, v: (v, 0)),
            pl.BlockSpec((1, VT), lambda p, v: (0, v)),
        ],
        out_specs=pl.BlockSpec((BATCH, VT), lambda p, v: (0, v * p)),
        out_shape=jax.ShapeDtypeStruct((BATCH, VOCAB), jnp.float32),
        scratch_shapes=[
            pltpu.VMEM((BATCH, 1), jnp.float32),
            pltpu.VMEM((BATCH, 1), jnp.float32),
            pltpu.VMEM((BATCH, 1), jnp.float32),
        ],
        interpret=interpret,
    )(s, W, b2)


def kernel(x, emb, W, b):
    xf = x.astype(jnp.int32).reshape(NW, IPW)
    xp = jnp.pad(xf, ((0, 0), (0, IPW_PAD - IPW))).reshape(NW, NCHUNK, 128)
    s = _sc_gather_sum(xp, emb)
    return _tc_logsoftmax(s, W, b.reshape(1, VOCAB))
